# initial kernel scaffold (unmeasured)
import jax
import jax.numpy as jnp
from jax import lax
from jax.experimental import pallas as pl
from jax.experimental.pallas import tpu as pltpu

N_DEV = 4
M = 4096
N = 8192
CM = M // N_DEV
TILE = 512
NT = CM // TILE


def _allreduce_body(partial_ref, out_ref, recv_ref, acc_ref, a_ref,
                    rs_send, rs_recv, ag_send, ag_recv, cp_sems, exit_sem):
    i = lax.axis_index("i")
    left = (i - 1) % N_DEV
    right = (i + 1) % N_DEV
    own = (i + 1) % N_DEV

    barrier = pltpu.get_barrier_semaphore()
    for nbr in (left, right):
        pl.semaphore_signal(barrier, inc=1, device_id=(nbr,),
                            device_id_type=pl.DeviceIdType.MESH)
    pl.semaphore_wait(barrier, 2)

    for s in range(N_DEV - 1):
        c_send = (i - s) % N_DEV
        c_recv = (i - s - 1) % N_DEV
        src = (partial_ref.at[pl.ds(c_send * CM, CM), :] if s == 0
               else acc_ref)
        rdma = pltpu.make_async_remote_copy(
            src_ref=src,
            dst_ref=recv_ref.at[s],
            send_sem=rs_send.at[s],
            recv_sem=rs_recv.at[s],
            device_id=(right,),
            device_id_type=pl.DeviceIdType.MESH,
        )
        rdma.start()
        rdma.wait()

        for t in range(NT):
            cp = pltpu.make_async_copy(
                partial_ref.at[pl.ds(c_recv * CM + t * TILE, TILE), :],
                a_ref,
                cp_sems.at[0],
            )
            cp.start()
            cp.wait()
            acc_ref[t * TILE:(t + 1) * TILE, :] = (
                a_ref[...] + recv_ref[s, t * TILE:(t + 1) * TILE, :]
            )

    st = pltpu.make_async_copy(
        acc_ref, out_ref.at[pl.ds(own * CM, CM), :], cp_sems.at[1])
    st.start()

    for t in range(N_DEV - 1):
        g = (own - t) % N_DEV
        src = acc_ref if t == 0 else out_ref.at[pl.ds(g * CM, CM), :]
        rdma = pltpu.make_async_remote_copy(
            src_ref=src,
            dst_ref=out_ref.at[pl.ds(g * CM, CM), :],
            send_sem=ag_send.at[t],
            recv_sem=ag_recv.at[t],
            device_id=(right,),
            device_id_type=pl.DeviceIdType.MESH,
        )
        rdma.start()
        rdma.wait()

    st.wait()

    for nbr in (left, right):
        pl.semaphore_signal(exit_sem, inc=1, device_id=(nbr,),
                            device_id_type=pl.DeviceIdType.MESH)
    pl.semaphore_wait(exit_sem, 2)


def _allreduce(partial):
    return pl.pallas_call(
        _allreduce_body,
        out_shape=jax.ShapeDtypeStruct((M, N), jnp.bfloat16),
        in_specs=[pl.BlockSpec(memory_space=pltpu.ANY)],
        out_specs=pl.BlockSpec(memory_space=pltpu.ANY),
        scratch_shapes=[
            pltpu.VMEM((N_DEV - 1, CM, N), jnp.bfloat16),
            pltpu.VMEM((CM, N), jnp.bfloat16),
            pltpu.VMEM((TILE, N), jnp.bfloat16),
            pltpu.SemaphoreType.DMA((N_DEV - 1,)),
            pltpu.SemaphoreType.DMA((N_DEV - 1,)),
            pltpu.SemaphoreType.DMA((N_DEV - 1,)),
            pltpu.SemaphoreType.DMA((N_DEV - 1,)),
            pltpu.SemaphoreType.DMA((2,)),
            pltpu.SemaphoreType.REGULAR,
        ],
        compiler_params=pltpu.CompilerParams(collective_id=0),
    )(partial)


def kernel(x, w_mat):
    partial = jnp.dot(
        x.astype(jnp.bfloat16), w_mat.astype(jnp.bfloat16),
        preferred_element_type=jnp.float32,
    ).astype(jnp.bfloat16)

    y = _allreduce(partial).astype(jnp.float32)

    amax = jnp.max(jnp.abs(y))
    scale = amax / 448.0
    q = (y / scale).astype(jnp.float8_e4m3fn).astype(jnp.float32)
    return q * scale


# baseline (device time: 1652704 ns/iter reference)
import jax
import jax.numpy as jnp
from jax import lax
from jax.experimental import pallas as pl
from jax.experimental.pallas import tpu as pltpu

N_DEV = 4
M = 4096
N = 8192
CM = M // N_DEV
TILE = 256
NT = CM // TILE


def _neighbor_barrier(sem, nbrs, count):
    for nbr in nbrs:
        pl.semaphore_signal(sem, inc=1, device_id=(nbr,),
                            device_id_type=pl.DeviceIdType.MESH)
    pl.semaphore_wait(sem, count)


def _rs_body(partial_ref, own_ref, recv_ref, acc_ref, a_ref, b_ref,
             rs_send, rs_recv, cp_sems, exit_sem):
    i = lax.axis_index("i")
    left = (i - 1) % N_DEV
    right = (i + 1) % N_DEV

    barrier = pltpu.get_barrier_semaphore()
    _neighbor_barrier(barrier, (left, right), 2)

    for s in range(N_DEV - 1):
        c_send = (i - s) % N_DEV
        c_recv = (i - s - 1) % N_DEV
        src = (partial_ref.at[pl.ds(c_send * CM, CM), :] if s == 0
               else acc_ref)
        rdma = pltpu.make_async_remote_copy(
            src_ref=src,
            dst_ref=recv_ref.at[s],
            send_sem=rs_send.at[s],
            recv_sem=rs_recv.at[s],
            device_id=(right,),
            device_id_type=pl.DeviceIdType.MESH,
        )
        rdma.start()
        rdma.wait()

        for t in range(NT):
            cp_a = pltpu.make_async_copy(
                partial_ref.at[pl.ds(c_recv * CM + t * TILE, TILE), :],
                a_ref,
                cp_sems.at[0],
            )
            cp_b = pltpu.make_async_copy(
                recv_ref.at[s, t * TILE:(t + 1) * TILE, :],
                b_ref,
                cp_sems.at[1],
            )
            cp_a.start()
            cp_b.start()
            cp_a.wait()
            cp_b.wait()
            acc_ref[t * TILE:(t + 1) * TILE, :] = a_ref[...] + b_ref[...]

    st = pltpu.make_async_copy(acc_ref, own_ref, cp_sems.at[2])
    st.start()
    st.wait()

    _neighbor_barrier(exit_sem, (left, right), 2)


def _reduce_scatter(partial):
    own, _ = pl.pallas_call(
        _rs_body,
        out_shape=(
            jax.ShapeDtypeStruct((CM, N), jnp.float32),
            jax.ShapeDtypeStruct((N_DEV - 1, CM, N), jnp.float32),
        ),
        in_specs=[pl.BlockSpec(memory_space=pltpu.HBM)],
        out_specs=(
            pl.BlockSpec(memory_space=pltpu.HBM),
            pl.BlockSpec(memory_space=pltpu.HBM),
        ),
        scratch_shapes=[
            pltpu.VMEM((CM, N), jnp.float32),
            pltpu.VMEM((TILE, N), jnp.float32),
            pltpu.VMEM((TILE, N), jnp.float32),
            pltpu.SemaphoreType.DMA((N_DEV - 1,)),
            pltpu.SemaphoreType.DMA((N_DEV - 1,)),
            pltpu.SemaphoreType.DMA((3,)),
            pltpu.SemaphoreType.REGULAR,
        ],
        compiler_params=pltpu.CompilerParams(
            collective_id=0, vmem_limit_bytes=60 * 1024 * 1024),
    )(partial)
    return own


def _amax_body(amax_ref, buf_ref, ax_send, ax_recv, exit_sem):
    i = lax.axis_index("i")
    peers = [(i + d) % N_DEV for d in (1, 2, 3)]

    barrier = pltpu.get_barrier_semaphore()
    _neighbor_barrier(barrier, peers, 3)

    buf_ref[pl.ds(i * 8, 8), :] = amax_ref[...]

    rdmas = []
    for d in (1, 2, 3):
        rdma = pltpu.make_async_remote_copy(
            src_ref=buf_ref.at[pl.ds(i * 8, 8), :],
            dst_ref=buf_ref.at[pl.ds(i * 8, 8), :],
            send_sem=ax_send.at[d - 1],
            recv_sem=ax_recv.at[d - 1],
            device_id=(peers[d - 1],),
            device_id_type=pl.DeviceIdType.MESH,
        )
        rdma.start()
        rdmas.append(rdma)
    for rdma in rdmas:
        rdma.wait()

    _neighbor_barrier(exit_sem, peers, 3)


def _amax_allgather(local_amax):
    buf = pl.pallas_call(
        _amax_body,
        out_shape=jax.ShapeDtypeStruct((N_DEV * 8, 128), jnp.float32),
        in_specs=[pl.BlockSpec(memory_space=pltpu.VMEM)],
        out_specs=pl.BlockSpec(memory_space=pltpu.VMEM),
        scratch_shapes=[
            pltpu.SemaphoreType.DMA((3,)),
            pltpu.SemaphoreType.DMA((3,)),
            pltpu.SemaphoreType.REGULAR,
        ],
        compiler_params=pltpu.CompilerParams(collective_id=1),
    )(local_amax)
    return jnp.max(buf)


def _ag_body(qchunk_ref, qfull_ref, ag_send, ag_recv, cp_sems, exit_sem):
    i = lax.axis_index("i")
    left = (i - 1) % N_DEV
    right = (i + 1) % N_DEV
    own = (i + 1) % N_DEV

    barrier = pltpu.get_barrier_semaphore()
    _neighbor_barrier(barrier, (left, right), 2)

    st = pltpu.make_async_copy(
        qchunk_ref, qfull_ref.at[pl.ds(own * CM, CM), :], cp_sems.at[0])
    st.start()

    for t in range(N_DEV - 1):
        g = (own - t) % N_DEV
        src = (qchunk_ref if t == 0
               else qfull_ref.at[pl.ds(g * CM, CM), :])
        rdma = pltpu.make_async_remote_copy(
            src_ref=src,
            dst_ref=qfull_ref.at[pl.ds(g * CM, CM), :],
            send_sem=ag_send.at[t],
            recv_sem=ag_recv.at[t],
            device_id=(right,),
            device_id_type=pl.DeviceIdType.MESH,
        )
        rdma.start()
        rdma.wait()

    st.wait()

    _neighbor_barrier(exit_sem, (left, right), 2)


def _quant_allgather(qchunk):
    return pl.pallas_call(
        _ag_body,
        out_shape=jax.ShapeDtypeStruct((M, N), qchunk.dtype),
        in_specs=[pl.BlockSpec(memory_space=pltpu.HBM)],
        out_specs=pl.BlockSpec(memory_space=pltpu.HBM),
        scratch_shapes=[
            pltpu.SemaphoreType.DMA((N_DEV - 1,)),
            pltpu.SemaphoreType.DMA((N_DEV - 1,)),
            pltpu.SemaphoreType.DMA((1,)),
            pltpu.SemaphoreType.REGULAR,
        ],
        compiler_params=pltpu.CompilerParams(collective_id=2),
    )(qchunk)


def kernel(x, w_mat):
    partial = jnp.dot(
        x.astype(jnp.bfloat16), w_mat.astype(jnp.bfloat16),
        preferred_element_type=jnp.float32,
    )

    own_chunk = _reduce_scatter(partial)

    local_amax = jnp.full((8, 128), jnp.max(jnp.abs(own_chunk)), jnp.float32)
    amax = _amax_allgather(local_amax)
    scale = amax / 448.0

    qchunk = (own_chunk / scale).astype(jnp.float8_e4m3fn)
    qfull = _quant_allgather(qchunk)

    return qfull.astype(jnp.float32) * scale


# device time: 988622 ns/iter; 1.6717x vs baseline; 1.6717x over previous
import jax
import jax.numpy as jnp
from jax import lax
from jax.experimental import pallas as pl
from jax.experimental.pallas import tpu as pltpu

N_DEV = 4
M = 4096
N = 8192
NL = N // 2
CM = M // N_DEV
TILE = 256
NT = CM // TILE


def _neighbor_barrier(sem, nbrs, count):
    for nbr in nbrs:
        pl.semaphore_signal(sem, inc=1, device_id=(nbr,),
                            device_id_type=pl.DeviceIdType.MESH)
    pl.semaphore_wait(sem, count)


def _rs_body(partial_ref, own_r_ref, own_l_ref, recv_r_ref, recv_l_ref,
             acc_r_ref, acc_l_ref, a_ref, b_ref,
             rs_send_r, rs_recv_r, rs_send_l, rs_recv_l, cp_sems, exit_sem):
    i = lax.axis_index("i")
    left = (i - 1) % N_DEV
    right = (i + 1) % N_DEV

    barrier = pltpu.get_barrier_semaphore()
    _neighbor_barrier(barrier, (left, right), 2)

    for s in range(N_DEV - 1):
        c_send_r = (i - s) % N_DEV
        c_recv_r = (i - s - 1) % N_DEV
        c_send_l = (i + s) % N_DEV
        c_recv_l = (i + s + 1) % N_DEV

        src_r = (partial_ref.at[pl.ds(c_send_r * CM, CM), pl.ds(0, NL)]
                 if s == 0 else acc_r_ref)
        rdma_r = pltpu.make_async_remote_copy(
            src_ref=src_r,
            dst_ref=recv_r_ref.at[s],
            send_sem=rs_send_r.at[s],
            recv_sem=rs_recv_r.at[s],
            device_id=(right,),
            device_id_type=pl.DeviceIdType.MESH,
        )
        src_l = (partial_ref.at[pl.ds(c_send_l * CM, CM), pl.ds(NL, NL)]
                 if s == 0 else acc_l_ref)
        rdma_l = pltpu.make_async_remote_copy(
            src_ref=src_l,
            dst_ref=recv_l_ref.at[s],
            send_sem=rs_send_l.at[s],
            recv_sem=rs_recv_l.at[s],
            device_id=(left,),
            device_id_type=pl.DeviceIdType.MESH,
        )
        rdma_r.start()
        rdma_l.start()

        rdma_r.wait()
        for t in range(NT):
            cp_a = pltpu.make_async_copy(
                partial_ref.at[pl.ds(c_recv_r * CM + t * TILE, TILE),
                               pl.ds(0, NL)],
                a_ref, cp_sems.at[0])
            cp_b = pltpu.make_async_copy(
                recv_r_ref.at[s, t * TILE:(t + 1) * TILE, :],
                b_ref, cp_sems.at[1])
            cp_a.start()
            cp_b.start()
            cp_a.wait()
            cp_b.wait()
            acc_r_ref[t * TILE:(t + 1) * TILE, :] = a_ref[...] + b_ref[...]

        rdma_l.wait()
        for t in range(NT):
            cp_a = pltpu.make_async_copy(
                partial_ref.at[pl.ds(c_recv_l * CM + t * TILE, TILE),
                               pl.ds(NL, NL)],
                a_ref, cp_sems.at[0])
            cp_b = pltpu.make_async_copy(
                recv_l_ref.at[s, t * TILE:(t + 1) * TILE, :],
                b_ref, cp_sems.at[1])
            cp_a.start()
            cp_b.start()
            cp_a.wait()
            cp_b.wait()
            acc_l_ref[t * TILE:(t + 1) * TILE, :] = a_ref[...] + b_ref[...]

    st_r = pltpu.make_async_copy(acc_r_ref, own_r_ref, cp_sems.at[2])
    st_l = pltpu.make_async_copy(acc_l_ref, own_l_ref, cp_sems.at[3])
    st_r.start()
    st_l.start()
    st_r.wait()
    st_l.wait()

    _neighbor_barrier(exit_sem, (left, right), 2)


def _reduce_scatter(partial):
    own_r, own_l, _, _ = pl.pallas_call(
        _rs_body,
        out_shape=(
            jax.ShapeDtypeStruct((CM, NL), jnp.float32),
            jax.ShapeDtypeStruct((CM, NL), jnp.float32),
            jax.ShapeDtypeStruct((N_DEV - 1, CM, NL), jnp.float32),
            jax.ShapeDtypeStruct((N_DEV - 1, CM, NL), jnp.float32),
        ),
        in_specs=[pl.BlockSpec(memory_space=pltpu.HBM)],
        out_specs=(
            pl.BlockSpec(memory_space=pltpu.HBM),
            pl.BlockSpec(memory_space=pltpu.HBM),
            pl.BlockSpec(memory_space=pltpu.HBM),
            pl.BlockSpec(memory_space=pltpu.HBM),
        ),
        scratch_shapes=[
            pltpu.VMEM((CM, NL), jnp.float32),
            pltpu.VMEM((CM, NL), jnp.float32),
            pltpu.VMEM((TILE, NL), jnp.float32),
            pltpu.VMEM((TILE, NL), jnp.float32),
            pltpu.SemaphoreType.DMA((N_DEV - 1,)),
            pltpu.SemaphoreType.DMA((N_DEV - 1,)),
            pltpu.SemaphoreType.DMA((N_DEV - 1,)),
            pltpu.SemaphoreType.DMA((N_DEV - 1,)),
            pltpu.SemaphoreType.DMA((4,)),
            pltpu.SemaphoreType.REGULAR,
        ],
        compiler_params=pltpu.CompilerParams(
            collective_id=0, vmem_limit_bytes=60 * 1024 * 1024),
    )(partial)
    return own_r, own_l


def _amax_body(amax_ref, buf_ref, ax_send, ax_recv, exit_sem):
    i = lax.axis_index("i")
    peers = [(i + d) % N_DEV for d in (1, 2, 3)]

    barrier = pltpu.get_barrier_semaphore()
    _neighbor_barrier(barrier, peers, 3)

    buf_ref[pl.ds(i * 8, 8), :] = amax_ref[...]

    rdmas = []
    for d in (1, 2, 3):
        rdma = pltpu.make_async_remote_copy(
            src_ref=buf_ref.at[pl.ds(i * 8, 8), :],
            dst_ref=buf_ref.at[pl.ds(i * 8, 8), :],
            send_sem=ax_send.at[d - 1],
            recv_sem=ax_recv.at[d - 1],
            device_id=(peers[d - 1],),
            device_id_type=pl.DeviceIdType.MESH,
        )
        rdma.start()
        rdmas.append(rdma)
    for rdma in rdmas:
        rdma.wait()

    _neighbor_barrier(exit_sem, peers, 3)


def _amax_allgather(local_amax):
    buf = pl.pallas_call(
        _amax_body,
        out_shape=jax.ShapeDtypeStruct((N_DEV * 8, 128), jnp.float32),
        in_specs=[pl.BlockSpec(memory_space=pltpu.VMEM)],
        out_specs=pl.BlockSpec(memory_space=pltpu.VMEM),
        scratch_shapes=[
            pltpu.SemaphoreType.DMA((3,)),
            pltpu.SemaphoreType.DMA((3,)),
            pltpu.SemaphoreType.REGULAR,
        ],
        compiler_params=pltpu.CompilerParams(collective_id=1),
    )(local_amax)
    return jnp.max(buf)


def _ag_body(q_r_ref, q_l_ref, qfull_ref, ag_send_r, ag_recv_r,
             ag_send_l, ag_recv_l, cp_sems, exit_sem):
    i = lax.axis_index("i")
    left = (i - 1) % N_DEV
    right = (i + 1) % N_DEV
    own_r = (i + 1) % N_DEV
    own_l = (i + 3) % N_DEV

    barrier = pltpu.get_barrier_semaphore()
    _neighbor_barrier(barrier, (left, right), 2)

    st_r = pltpu.make_async_copy(
        q_r_ref, qfull_ref.at[pl.ds(own_r * CM, CM), pl.ds(0, NL)],
        cp_sems.at[0])
    st_l = pltpu.make_async_copy(
        q_l_ref, qfull_ref.at[pl.ds(own_l * CM, CM), pl.ds(NL, NL)],
        cp_sems.at[1])
    st_r.start()
    st_l.start()

    for t in range(N_DEV - 1):
        g_r = (own_r - t) % N_DEV
        g_l = (own_l + t) % N_DEV
        src_r = (q_r_ref if t == 0
                 else qfull_ref.at[pl.ds(g_r * CM, CM), pl.ds(0, NL)])
        rdma_r = pltpu.make_async_remote_copy(
            src_ref=src_r,
            dst_ref=qfull_ref.at[pl.ds(g_r * CM, CM), pl.ds(0, NL)],
            send_sem=ag_send_r.at[t],
            recv_sem=ag_recv_r.at[t],
            device_id=(right,),
            device_id_type=pl.DeviceIdType.MESH,
        )
        src_l = (q_l_ref if t == 0
                 else qfull_ref.at[pl.ds(g_l * CM, CM), pl.ds(NL, NL)])
        rdma_l = pltpu.make_async_remote_copy(
            src_ref=src_l,
            dst_ref=qfull_ref.at[pl.ds(g_l * CM, CM), pl.ds(NL, NL)],
            send_sem=ag_send_l.at[t],
            recv_sem=ag_recv_l.at[t],
            device_id=(left,),
            device_id_type=pl.DeviceIdType.MESH,
        )
        rdma_r.start()
        rdma_l.start()
        rdma_r.wait()
        rdma_l.wait()

    st_r.wait()
    st_l.wait()

    _neighbor_barrier(exit_sem, (left, right), 2)


def _quant_allgather(q_r, q_l):
    return pl.pallas_call(
        _ag_body,
        out_shape=jax.ShapeDtypeStruct((M, N), q_r.dtype),
        in_specs=[
            pl.BlockSpec(memory_space=pltpu.HBM),
            pl.BlockSpec(memory_space=pltpu.HBM),
        ],
        out_specs=pl.BlockSpec(memory_space=pltpu.HBM),
        scratch_shapes=[
            pltpu.SemaphoreType.DMA((N_DEV - 1,)),
            pltpu.SemaphoreType.DMA((N_DEV - 1,)),
            pltpu.SemaphoreType.DMA((N_DEV - 1,)),
            pltpu.SemaphoreType.DMA((N_DEV - 1,)),
            pltpu.SemaphoreType.DMA((2,)),
            pltpu.SemaphoreType.REGULAR,
        ],
        compiler_params=pltpu.CompilerParams(collective_id=2),
    )(q_r, q_l)


def kernel(x, w_mat):
    partial = jnp.dot(
        x.astype(jnp.bfloat16), w_mat.astype(jnp.bfloat16),
        preferred_element_type=jnp.float32,
    )

    own_r, own_l = _reduce_scatter(partial)

    local = jnp.maximum(jnp.max(jnp.abs(own_r)), jnp.max(jnp.abs(own_l)))
    amax = _amax_allgather(jnp.full((8, 128), local, jnp.float32))
    scale = amax / 448.0

    q_r = (own_r / scale).astype(jnp.float8_e4m3fn)
    q_l = (own_l / scale).astype(jnp.float8_e4m3fn)
    qfull = _quant_allgather(q_r, q_l)

    return qfull.astype(jnp.float32) * scale


# device time: 937130 ns/iter; 1.7636x vs baseline; 1.0549x over previous
import jax
import jax.numpy as jnp
from jax import lax
from jax.experimental import pallas as pl
from jax.experimental.pallas import tpu as pltpu

N_DEV = 4
M = 4096
N = 8192
NL = N // 2
CM = M // N_DEV
TILE = 256
NT = CM // TILE


def _neighbor_barrier(sem, nbrs, count):
    for nbr in nbrs:
        pl.semaphore_signal(sem, inc=1, device_id=(nbr,),
                            device_id_type=pl.DeviceIdType.MESH)
    pl.semaphore_wait(sem, count)


def _rs_body(partial_ref, own_r_ref, own_l_ref, recv_r_ref, recv_l_ref,
             acc_r_ref, acc_l_ref, a_ref, b_ref,
             rs_send_r, rs_recv_r, rs_send_l, rs_recv_l, cp_sems, exit_sem):
    i = lax.axis_index("i")
    left = (i - 1) % N_DEV
    right = (i + 1) % N_DEV

    barrier = pltpu.get_barrier_semaphore()
    _neighbor_barrier(barrier, (left, right), 2)

    def _make_rdma(src, dst_ref, send_sems, recv_sems, s, dev):
        return pltpu.make_async_remote_copy(
            src_ref=src,
            dst_ref=dst_ref.at[s],
            send_sem=send_sems.at[s],
            recv_sem=recv_sems.at[s],
            device_id=(dev,),
            device_id_type=pl.DeviceIdType.MESH,
        )

    def _adds(s, c_recv, col0, recv_ref, acc_ref):
        for t in range(NT):
            cp_a = pltpu.make_async_copy(
                partial_ref.at[pl.ds(c_recv * CM + t * TILE, TILE),
                               pl.ds(col0, NL)],
                a_ref, cp_sems.at[0])
            cp_b = pltpu.make_async_copy(
                recv_ref.at[s, t * TILE:(t + 1) * TILE, :],
                b_ref, cp_sems.at[1])
            cp_a.start()
            cp_b.start()
            cp_a.wait()
            cp_b.wait()
            acc_ref[t * TILE:(t + 1) * TILE, :] = a_ref[...] + b_ref[...]

    rdma_r = _make_rdma(
        partial_ref.at[pl.ds(i * CM, CM), pl.ds(0, NL)],
        recv_r_ref, rs_send_r, rs_recv_r, 0, right)
    rdma_l = _make_rdma(
        partial_ref.at[pl.ds(i * CM, CM), pl.ds(NL, NL)],
        recv_l_ref, rs_send_l, rs_recv_l, 0, left)
    rdma_r.start()
    rdma_l.start()

    for s in range(N_DEV - 1):
        rdma_r.wait()
        _adds(s, (i - s - 1) % N_DEV, 0, recv_r_ref, acc_r_ref)
        if s < N_DEV - 2:
            next_r = _make_rdma(
                acc_r_ref, recv_r_ref, rs_send_r, rs_recv_r, s + 1, right)
            next_r.start()
        rdma_l.wait()
        _adds(s, (i + s + 1) % N_DEV, NL, recv_l_ref, acc_l_ref)
        if s < N_DEV - 2:
            next_l = _make_rdma(
                acc_l_ref, recv_l_ref, rs_send_l, rs_recv_l, s + 1, left)
            next_l.start()
            rdma_r, rdma_l = next_r, next_l

    st_r = pltpu.make_async_copy(acc_r_ref, own_r_ref, cp_sems.at[2])
    st_l = pltpu.make_async_copy(acc_l_ref, own_l_ref, cp_sems.at[3])
    st_r.start()
    st_l.start()
    st_r.wait()
    st_l.wait()

    _neighbor_barrier(exit_sem, (left, right), 2)


def _reduce_scatter(partial):
    own_r, own_l, _, _ = pl.pallas_call(
        _rs_body,
        out_shape=(
            jax.ShapeDtypeStruct((CM, NL), jnp.float32),
            jax.ShapeDtypeStruct((CM, NL), jnp.float32),
            jax.ShapeDtypeStruct((N_DEV - 1, CM, NL), jnp.float32),
            jax.ShapeDtypeStruct((N_DEV - 1, CM, NL), jnp.float32),
        ),
        in_specs=[pl.BlockSpec(memory_space=pltpu.HBM)],
        out_specs=(
            pl.BlockSpec(memory_space=pltpu.HBM),
            pl.BlockSpec(memory_space=pltpu.HBM),
            pl.BlockSpec(memory_space=pltpu.HBM),
            pl.BlockSpec(memory_space=pltpu.HBM),
        ),
        scratch_shapes=[
            pltpu.VMEM((CM, NL), jnp.float32),
            pltpu.VMEM((CM, NL), jnp.float32),
            pltpu.VMEM((TILE, NL), jnp.float32),
            pltpu.VMEM((TILE, NL), jnp.float32),
            pltpu.SemaphoreType.DMA((N_DEV - 1,)),
            pltpu.SemaphoreType.DMA((N_DEV - 1,)),
            pltpu.SemaphoreType.DMA((N_DEV - 1,)),
            pltpu.SemaphoreType.DMA((N_DEV - 1,)),
            pltpu.SemaphoreType.DMA((4,)),
            pltpu.SemaphoreType.REGULAR,
        ],
        compiler_params=pltpu.CompilerParams(
            collective_id=0, vmem_limit_bytes=60 * 1024 * 1024),
    )(partial)
    return own_r, own_l


def _amax_body(amax_ref, buf_ref, ax_send, ax_recv, exit_sem):
    i = lax.axis_index("i")
    peers = [(i + d) % N_DEV for d in (1, 2, 3)]

    barrier = pltpu.get_barrier_semaphore()
    _neighbor_barrier(barrier, peers, 3)

    buf_ref[pl.ds(i * 8, 8), :] = amax_ref[...]

    rdmas = []
    for d in (1, 2, 3):
        rdma = pltpu.make_async_remote_copy(
            src_ref=buf_ref.at[pl.ds(i * 8, 8), :],
            dst_ref=buf_ref.at[pl.ds(i * 8, 8), :],
            send_sem=ax_send.at[d - 1],
            recv_sem=ax_recv.at[d - 1],
            device_id=(peers[d - 1],),
            device_id_type=pl.DeviceIdType.MESH,
        )
        rdma.start()
        rdmas.append(rdma)
    for rdma in rdmas:
        rdma.wait()

    _neighbor_barrier(exit_sem, peers, 3)


def _amax_allgather(local_amax):
    buf = pl.pallas_call(
        _amax_body,
        out_shape=jax.ShapeDtypeStruct((N_DEV * 8, 128), jnp.float32),
        in_specs=[pl.BlockSpec(memory_space=pltpu.VMEM)],
        out_specs=pl.BlockSpec(memory_space=pltpu.VMEM),
        scratch_shapes=[
            pltpu.SemaphoreType.DMA((3,)),
            pltpu.SemaphoreType.DMA((3,)),
            pltpu.SemaphoreType.REGULAR,
        ],
        compiler_params=pltpu.CompilerParams(collective_id=1),
    )(local_amax)
    return jnp.max(buf)


def _ag_body(q_r_ref, q_l_ref, qfull_ref, ag_send_r, ag_recv_r,
             ag_send_l, ag_recv_l, cp_sems, exit_sem):
    i = lax.axis_index("i")
    left = (i - 1) % N_DEV
    right = (i + 1) % N_DEV
    own_r = (i + 1) % N_DEV
    own_l = (i + 3) % N_DEV

    barrier = pltpu.get_barrier_semaphore()
    _neighbor_barrier(barrier, (left, right), 2)

    st_r = pltpu.make_async_copy(
        q_r_ref, qfull_ref.at[pl.ds(own_r * CM, CM), pl.ds(0, NL)],
        cp_sems.at[0])
    st_l = pltpu.make_async_copy(
        q_l_ref, qfull_ref.at[pl.ds(own_l * CM, CM), pl.ds(NL, NL)],
        cp_sems.at[1])
    st_r.start()
    st_l.start()

    for t in range(N_DEV - 1):
        g_r = (own_r - t) % N_DEV
        g_l = (own_l + t) % N_DEV
        src_r = (q_r_ref if t == 0
                 else qfull_ref.at[pl.ds(g_r * CM, CM), pl.ds(0, NL)])
        rdma_r = pltpu.make_async_remote_copy(
            src_ref=src_r,
            dst_ref=qfull_ref.at[pl.ds(g_r * CM, CM), pl.ds(0, NL)],
            send_sem=ag_send_r.at[t],
            recv_sem=ag_recv_r.at[t],
            device_id=(right,),
            device_id_type=pl.DeviceIdType.MESH,
        )
        src_l = (q_l_ref if t == 0
                 else qfull_ref.at[pl.ds(g_l * CM, CM), pl.ds(NL, NL)])
        rdma_l = pltpu.make_async_remote_copy(
            src_ref=src_l,
            dst_ref=qfull_ref.at[pl.ds(g_l * CM, CM), pl.ds(NL, NL)],
            send_sem=ag_send_l.at[t],
            recv_sem=ag_recv_l.at[t],
            device_id=(left,),
            device_id_type=pl.DeviceIdType.MESH,
        )
        rdma_r.start()
        rdma_l.start()
        rdma_r.wait()
        rdma_l.wait()

    st_r.wait()
    st_l.wait()

    _neighbor_barrier(exit_sem, (left, right), 2)


def _quant_allgather(q_r, q_l):
    return pl.pallas_call(
        _ag_body,
        out_shape=jax.ShapeDtypeStruct((M, N), q_r.dtype),
        in_specs=[
            pl.BlockSpec(memory_space=pltpu.HBM),
            pl.BlockSpec(memory_space=pltpu.HBM),
        ],
        out_specs=pl.BlockSpec(memory_space=pltpu.HBM),
        scratch_shapes=[
            pltpu.SemaphoreType.DMA((N_DEV - 1,)),
            pltpu.SemaphoreType.DMA((N_DEV - 1,)),
            pltpu.SemaphoreType.DMA((N_DEV - 1,)),
            pltpu.SemaphoreType.DMA((N_DEV - 1,)),
            pltpu.SemaphoreType.DMA((2,)),
            pltpu.SemaphoreType.REGULAR,
        ],
        compiler_params=pltpu.CompilerParams(collective_id=2),
    )(q_r, q_l)


def kernel(x, w_mat):
    partial = jnp.dot(
        x.astype(jnp.bfloat16), w_mat.astype(jnp.bfloat16),
        preferred_element_type=jnp.float32,
    )

    own_r, own_l = _reduce_scatter(partial)

    local = jnp.maximum(jnp.max(jnp.abs(own_r)), jnp.max(jnp.abs(own_l)))
    amax = _amax_allgather(jnp.full((8, 128), local, jnp.float32))
    scale = amax / 448.0

    q_r = (own_r / scale).astype(jnp.float8_e4m3fn)
    q_l = (own_l / scale).astype(jnp.float8_e4m3fn)
    qfull = _quant_allgather(q_r, q_l)

    return (qfull.astype(jnp.float32) * scale).astype(jnp.bfloat16)


# device time: 888638 ns/iter; 1.8598x vs baseline; 1.0546x over previous
import jax
import jax.numpy as jnp
from jax import lax
from jax.experimental import pallas as pl
from jax.experimental.pallas import tpu as pltpu

N_DEV = 4
M = 4096
N = 8192
NL = N // 2
CM = M // N_DEV
TILE = 256
NT = CM // TILE


def _neighbor_barrier(sem, nbrs, count):
    for nbr in nbrs:
        pl.semaphore_signal(sem, inc=1, device_id=(nbr,),
                            device_id_type=pl.DeviceIdType.MESH)
    pl.semaphore_wait(sem, count)


KS = 1024


def _rs_body(x_ref, w_ref, own_r_ref, own_l_ref, amax_ref,
             recv_r_ref, recv_l_ref,
             w_vmem, x_tile, acc_r_ref, acc_l_ref, b_ref,
             rs_send_r, rs_recv_r, rs_send_l, rs_recv_l, cp_sems, exit_sem):
    i = lax.axis_index("i")
    left = (i - 1) % N_DEV
    right = (i + 1) % N_DEV

    wcp = pltpu.make_async_copy(w_ref, w_vmem, cp_sems.at[2])
    wcp.start()

    barrier = pltpu.get_barrier_semaphore()
    _neighbor_barrier(barrier, (left, right), 2)

    wcp.wait()

    def _make_rdma(src, dst_ref, send_sems, recv_sems, s, dev):
        return pltpu.make_async_remote_copy(
            src_ref=src,
            dst_ref=dst_ref.at[s],
            send_sem=send_sems.at[s],
            recv_sem=recv_sems.at[s],
            device_id=(dev,),
            device_id_type=pl.DeviceIdType.MESH,
        )

    def _partial_tile(c, t, col0):
        xcp = pltpu.make_async_copy(
            x_ref.at[pl.ds(c * CM + t * TILE, TILE), :],
            x_tile, cp_sems.at[0])
        xcp.start()
        xcp.wait()
        return jnp.dot(x_tile[...], w_vmem[:, col0:col0 + NL],
                       preferred_element_type=jnp.float32)

    for t in range(NT):
        acc_r_ref[t * TILE:(t + 1) * TILE, :] = _partial_tile(i, t, 0)
    rdma_r = _make_rdma(acc_r_ref, recv_r_ref, rs_send_r, rs_recv_r, 0, right)
    rdma_r.start()
    for t in range(NT):
        acc_l_ref[t * TILE:(t + 1) * TILE, :] = _partial_tile(i, t, NL)
    rdma_l = _make_rdma(acc_l_ref, recv_l_ref, rs_send_l, rs_recv_l, 0, left)
    rdma_l.start()

    amax = jnp.full((), 0.0, jnp.float32)

    def _adds(s, c_recv, col0, recv_ref, acc_ref, amax):
        for t in range(NT):
            cp_b = pltpu.make_async_copy(
                recv_ref.at[s, t * TILE:(t + 1) * TILE, :],
                b_ref, cp_sems.at[1])
            cp_b.start()
            v = _partial_tile(c_recv, t, col0)
            cp_b.wait()
            v = v + b_ref[...]
            acc_ref[t * TILE:(t + 1) * TILE, :] = v
            if s == N_DEV - 2:
                amax = jnp.maximum(amax, jnp.max(jnp.abs(v)))
        return amax

    for s in range(N_DEV - 1):
        rdma_r.wait()
        amax = _adds(s, (i - s - 1) % N_DEV, 0, recv_r_ref, acc_r_ref, amax)
        if s < N_DEV - 2:
            next_r = _make_rdma(
                acc_r_ref, recv_r_ref, rs_send_r, rs_recv_r, s + 1, right)
            next_r.start()
        rdma_l.wait()
        amax = _adds(s, (i + s + 1) % N_DEV, NL, recv_l_ref, acc_l_ref, amax)
        if s < N_DEV - 2:
            next_l = _make_rdma(
                acc_l_ref, recv_l_ref, rs_send_l, rs_recv_l, s + 1, left)
            next_l.start()
            rdma_r, rdma_l = next_r, next_l

    st_r = pltpu.make_async_copy(acc_r_ref, own_r_ref, cp_sems.at[2])
    st_l = pltpu.make_async_copy(acc_l_ref, own_l_ref, cp_sems.at[3])
    st_r.start()
    st_l.start()
    amax_ref[...] = jnp.full((8, 128), amax, jnp.float32)
    st_r.wait()
    st_l.wait()

    _neighbor_barrier(exit_sem, (left, right), 2)


def _reduce_scatter(x_bf16, w_bf16):
    own_r, own_l, amax, _, _ = pl.pallas_call(
        _rs_body,
        out_shape=(
            jax.ShapeDtypeStruct((CM, NL), jnp.float32),
            jax.ShapeDtypeStruct((CM, NL), jnp.float32),
            jax.ShapeDtypeStruct((8, 128), jnp.float32),
            jax.ShapeDtypeStruct((N_DEV - 1, CM, NL), jnp.float32),
            jax.ShapeDtypeStruct((N_DEV - 1, CM, NL), jnp.float32),
        ),
        in_specs=[
            pl.BlockSpec(memory_space=pltpu.HBM),
            pl.BlockSpec(memory_space=pltpu.HBM),
        ],
        out_specs=(
            pl.BlockSpec(memory_space=pltpu.HBM),
            pl.BlockSpec(memory_space=pltpu.HBM),
            pl.BlockSpec(memory_space=pltpu.VMEM),
            pl.BlockSpec(memory_space=pltpu.HBM),
            pl.BlockSpec(memory_space=pltpu.HBM),
        ),
        scratch_shapes=[
            pltpu.VMEM((KS, N), jnp.bfloat16),
            pltpu.VMEM((TILE, KS), jnp.bfloat16),
            pltpu.VMEM((CM, NL), jnp.float32),
            pltpu.VMEM((CM, NL), jnp.float32),
            pltpu.VMEM((TILE, NL), jnp.float32),
            pltpu.SemaphoreType.DMA((N_DEV - 1,)),
            pltpu.SemaphoreType.DMA((N_DEV - 1,)),
            pltpu.SemaphoreType.DMA((N_DEV - 1,)),
            pltpu.SemaphoreType.DMA((N_DEV - 1,)),
            pltpu.SemaphoreType.DMA((4,)),
            pltpu.SemaphoreType.REGULAR,
        ],
        compiler_params=pltpu.CompilerParams(
            collective_id=0, vmem_limit_bytes=62 * 1024 * 1024),
    )(x_bf16, w_bf16)
    return own_r, own_l, amax


def _amax_body(amax_ref, buf_ref, ax_send, ax_recv, exit_sem):
    i = lax.axis_index("i")
    peers = [(i + d) % N_DEV for d in (1, 2, 3)]

    barrier = pltpu.get_barrier_semaphore()
    _neighbor_barrier(barrier, peers, 3)

    buf_ref[pl.ds(i * 8, 8), :] = amax_ref[...]

    rdmas = []
    for d in (1, 2, 3):
        rdma = pltpu.make_async_remote_copy(
            src_ref=buf_ref.at[pl.ds(i * 8, 8), :],
            dst_ref=buf_ref.at[pl.ds(i * 8, 8), :],
            send_sem=ax_send.at[d - 1],
            recv_sem=ax_recv.at[d - 1],
            device_id=(peers[d - 1],),
            device_id_type=pl.DeviceIdType.MESH,
        )
        rdma.start()
        rdmas.append(rdma)
    for rdma in rdmas:
        rdma.wait()

    _neighbor_barrier(exit_sem, peers, 3)


def _amax_allgather(local_amax):
    buf = pl.pallas_call(
        _amax_body,
        out_shape=jax.ShapeDtypeStruct((N_DEV * 8, 128), jnp.float32),
        in_specs=[pl.BlockSpec(memory_space=pltpu.VMEM)],
        out_specs=pl.BlockSpec(memory_space=pltpu.VMEM),
        scratch_shapes=[
            pltpu.SemaphoreType.DMA((3,)),
            pltpu.SemaphoreType.DMA((3,)),
            pltpu.SemaphoreType.REGULAR,
        ],
        compiler_params=pltpu.CompilerParams(collective_id=1),
    )(local_amax)
    return jnp.max(buf)


def _ag_body(q_r_ref, q_l_ref, qfull_ref, ag_send_r, ag_recv_r,
             ag_send_l, ag_recv_l, cp_sems, exit_sem):
    i = lax.axis_index("i")
    left = (i - 1) % N_DEV
    right = (i + 1) % N_DEV
    own_r = (i + 1) % N_DEV
    own_l = (i + 3) % N_DEV

    barrier = pltpu.get_barrier_semaphore()
    _neighbor_barrier(barrier, (left, right), 2)

    st_r = pltpu.make_async_copy(
        q_r_ref, qfull_ref.at[pl.ds(own_r * CM, CM), pl.ds(0, NL)],
        cp_sems.at[0])
    st_l = pltpu.make_async_copy(
        q_l_ref, qfull_ref.at[pl.ds(own_l * CM, CM), pl.ds(NL, NL)],
        cp_sems.at[1])
    st_r.start()
    st_l.start()

    for t in range(N_DEV - 1):
        g_r = (own_r - t) % N_DEV
        g_l = (own_l + t) % N_DEV
        src_r = (q_r_ref if t == 0
                 else qfull_ref.at[pl.ds(g_r * CM, CM), pl.ds(0, NL)])
        rdma_r = pltpu.make_async_remote_copy(
            src_ref=src_r,
            dst_ref=qfull_ref.at[pl.ds(g_r * CM, CM), pl.ds(0, NL)],
            send_sem=ag_send_r.at[t],
            recv_sem=ag_recv_r.at[t],
            device_id=(right,),
            device_id_type=pl.DeviceIdType.MESH,
        )
        src_l = (q_l_ref if t == 0
                 else qfull_ref.at[pl.ds(g_l * CM, CM), pl.ds(NL, NL)])
        rdma_l = pltpu.make_async_remote_copy(
            src_ref=src_l,
            dst_ref=qfull_ref.at[pl.ds(g_l * CM, CM), pl.ds(NL, NL)],
            send_sem=ag_send_l.at[t],
            recv_sem=ag_recv_l.at[t],
            device_id=(left,),
            device_id_type=pl.DeviceIdType.MESH,
        )
        rdma_r.start()
        rdma_l.start()
        rdma_r.wait()
        rdma_l.wait()

    st_r.wait()
    st_l.wait()

    _neighbor_barrier(exit_sem, (left, right), 2)


def _quant_allgather(q_r, q_l):
    return pl.pallas_call(
        _ag_body,
        out_shape=jax.ShapeDtypeStruct((M, N), q_r.dtype),
        in_specs=[
            pl.BlockSpec(memory_space=pltpu.HBM),
            pl.BlockSpec(memory_space=pltpu.HBM),
        ],
        out_specs=pl.BlockSpec(memory_space=pltpu.HBM),
        scratch_shapes=[
            pltpu.SemaphoreType.DMA((N_DEV - 1,)),
            pltpu.SemaphoreType.DMA((N_DEV - 1,)),
            pltpu.SemaphoreType.DMA((N_DEV - 1,)),
            pltpu.SemaphoreType.DMA((N_DEV - 1,)),
            pltpu.SemaphoreType.DMA((2,)),
            pltpu.SemaphoreType.REGULAR,
        ],
        compiler_params=pltpu.CompilerParams(collective_id=2),
    )(q_r, q_l)


def kernel(x, w_mat):
    own_r, own_l, local_amax = _reduce_scatter(
        x.astype(jnp.bfloat16), w_mat.astype(jnp.bfloat16))

    amax = _amax_allgather(local_amax)
    scale = amax / 448.0

    q_r = (own_r / scale).astype(jnp.float8_e4m3fn)
    q_l = (own_l / scale).astype(jnp.float8_e4m3fn)
    qfull = _quant_allgather(q_r, q_l)

    return (qfull.astype(jnp.float32) * scale).astype(jnp.bfloat16)


# device time: 790648 ns/iter; 2.0903x vs baseline; 1.1239x over previous
import jax
import jax.numpy as jnp
from jax import lax
from jax.experimental import pallas as pl
from jax.experimental.pallas import tpu as pltpu

N_DEV = 4
M = 4096
N = 8192
NL = N // 2
CM = M // N_DEV
TILE = 256
NT = CM // TILE


def _neighbor_barrier(sem, nbrs, count):
    for nbr in nbrs:
        pl.semaphore_signal(sem, inc=1, device_id=(nbr,),
                            device_id_type=pl.DeviceIdType.MESH)
    pl.semaphore_wait(sem, count)


KS = 1024


def _rs_body(x_ref, w_ref, own_r_ref, own_l_ref, amax_ref,
             recv_r_ref, recv_l_ref,
             w_vmem, x_tile, acc_r_ref, acc_l_ref, b_ref,
             rs_send_r, rs_recv_r, rs_send_l, rs_recv_l, st_sems, cp_sems,
             exit_sem):
    i = lax.axis_index("i")
    left = (i - 1) % N_DEV
    right = (i + 1) % N_DEV

    wcp = pltpu.make_async_copy(w_ref, w_vmem, cp_sems.at[3])
    wcp.start()

    barrier = pltpu.get_barrier_semaphore()
    _neighbor_barrier(barrier, (left, right), 2)

    wcp.wait()

    def _ts(t):
        return slice(t * TILE, (t + 1) * TILE)

    def _send_tile(acc_ref, recv_ref, send_sems, recv_sems, s, t, dev):
        r = pltpu.make_async_remote_copy(
            src_ref=acc_ref.at[_ts(t), :],
            dst_ref=recv_ref.at[s, _ts(t), :],
            send_sem=send_sems.at[s * NT + t],
            recv_sem=recv_sems.at[s * NT + t],
            device_id=(dev,),
            device_id_type=pl.DeviceIdType.MESH,
        )
        r.start()
        return r

    def _partial_tile(c, t, col0):
        xcp = pltpu.make_async_copy(
            x_ref.at[pl.ds(c * CM + t * TILE, TILE), :],
            x_tile, cp_sems.at[0])
        xcp.start()
        xcp.wait()
        return jnp.dot(x_tile[...], w_vmem[:, col0:col0 + NL],
                       preferred_element_type=jnp.float32)

    rd_r = [None] * NT
    rd_l = [None] * NT
    for t in range(NT):
        acc_r_ref[_ts(t), :] = _partial_tile(i, t, 0)
        rd_r[t] = _send_tile(acc_r_ref, recv_r_ref, rs_send_r, rs_recv_r,
                             0, t, right)
        acc_l_ref[_ts(t), :] = _partial_tile(i, t, NL)
        rd_l[t] = _send_tile(acc_l_ref, recv_l_ref, rs_send_l, rs_recv_l,
                             0, t, left)

    amax = jnp.full((), 0.0, jnp.float32)
    sts = []

    for s in range(N_DEV - 1):
        c_r = (i - s - 1) % N_DEV
        c_l = (i + s + 1) % N_DEV
        final = s == N_DEV - 2
        for t in range(NT):
            rd_r[t].wait()
            cp_b = pltpu.make_async_copy(
                recv_r_ref.at[s, _ts(t), :], b_ref, cp_sems.at[1])
            cp_b.start()
            v = _partial_tile(c_r, t, 0)
            cp_b.wait()
            v = v + b_ref[...]
            acc_r_ref[_ts(t), :] = v
            if final:
                amax = jnp.maximum(amax, jnp.max(jnp.abs(v)))
                st = pltpu.make_async_copy(
                    acc_r_ref.at[_ts(t), :], own_r_ref.at[_ts(t), :],
                    st_sems.at[t])
                st.start()
                sts.append(st)
            else:
                rd_r[t] = _send_tile(acc_r_ref, recv_r_ref, rs_send_r,
                                     rs_recv_r, s + 1, t, right)

            rd_l[t].wait()
            cp_b2 = pltpu.make_async_copy(
                recv_l_ref.at[s, _ts(t), :], b_ref, cp_sems.at[2])
            cp_b2.start()
            v2 = _partial_tile(c_l, t, NL)
            cp_b2.wait()
            v2 = v2 + b_ref[...]
            acc_l_ref[_ts(t), :] = v2
            if final:
                amax = jnp.maximum(amax, jnp.max(jnp.abs(v2)))
                st = pltpu.make_async_copy(
                    acc_l_ref.at[_ts(t), :], own_l_ref.at[_ts(t), :],
                    st_sems.at[NT + t])
                st.start()
                sts.append(st)
            else:
                rd_l[t] = _send_tile(acc_l_ref, recv_l_ref, rs_send_l,
                                     rs_recv_l, s + 1, t, left)

    amax_ref[...] = jnp.full((8, 128), amax, jnp.float32)
    for st in sts:
        st.wait()

    _neighbor_barrier(exit_sem, (left, right), 2)


def _reduce_scatter(x_bf16, w_bf16):
    own_r, own_l, amax, _, _ = pl.pallas_call(
        _rs_body,
        out_shape=(
            jax.ShapeDtypeStruct((CM, NL), jnp.float32),
            jax.ShapeDtypeStruct((CM, NL), jnp.float32),
            jax.ShapeDtypeStruct((8, 128), jnp.float32),
            jax.ShapeDtypeStruct((N_DEV - 1, CM, NL), jnp.float32),
            jax.ShapeDtypeStruct((N_DEV - 1, CM, NL), jnp.float32),
        ),
        in_specs=[
            pl.BlockSpec(memory_space=pltpu.HBM),
            pl.BlockSpec(memory_space=pltpu.HBM),
        ],
        out_specs=(
            pl.BlockSpec(memory_space=pltpu.HBM),
            pl.BlockSpec(memory_space=pltpu.HBM),
            pl.BlockSpec(memory_space=pltpu.VMEM),
            pl.BlockSpec(memory_space=pltpu.HBM),
            pl.BlockSpec(memory_space=pltpu.HBM),
        ),
        scratch_shapes=[
            pltpu.VMEM((KS, N), jnp.bfloat16),
            pltpu.VMEM((TILE, KS), jnp.bfloat16),
            pltpu.VMEM((CM, NL), jnp.float32),
            pltpu.VMEM((CM, NL), jnp.float32),
            pltpu.VMEM((TILE, NL), jnp.float32),
            pltpu.SemaphoreType.DMA(((N_DEV - 1) * NT,)),
            pltpu.SemaphoreType.DMA(((N_DEV - 1) * NT,)),
            pltpu.SemaphoreType.DMA(((N_DEV - 1) * NT,)),
            pltpu.SemaphoreType.DMA(((N_DEV - 1) * NT,)),
            pltpu.SemaphoreType.DMA((2 * NT,)),
            pltpu.SemaphoreType.DMA((4,)),
            pltpu.SemaphoreType.REGULAR,
        ],
        compiler_params=pltpu.CompilerParams(
            collective_id=0, vmem_limit_bytes=62 * 1024 * 1024),
    )(x_bf16, w_bf16)
    return own_r, own_l, amax


def _amax_body(amax_ref, buf_ref, ax_send, ax_recv, exit_sem):
    i = lax.axis_index("i")
    peers = [(i + d) % N_DEV for d in (1, 2, 3)]

    barrier = pltpu.get_barrier_semaphore()
    _neighbor_barrier(barrier, peers, 3)

    buf_ref[pl.ds(i * 8, 8), :] = amax_ref[...]

    rdmas = []
    for d in (1, 2, 3):
        rdma = pltpu.make_async_remote_copy(
            src_ref=buf_ref.at[pl.ds(i * 8, 8), :],
            dst_ref=buf_ref.at[pl.ds(i * 8, 8), :],
            send_sem=ax_send.at[d - 1],
            recv_sem=ax_recv.at[d - 1],
            device_id=(peers[d - 1],),
            device_id_type=pl.DeviceIdType.MESH,
        )
        rdma.start()
        rdmas.append(rdma)
    for rdma in rdmas:
        rdma.wait()

    _neighbor_barrier(exit_sem, peers, 3)


def _amax_allgather(local_amax):
    buf = pl.pallas_call(
        _amax_body,
        out_shape=jax.ShapeDtypeStruct((N_DEV * 8, 128), jnp.float32),
        in_specs=[pl.BlockSpec(memory_space=pltpu.VMEM)],
        out_specs=pl.BlockSpec(memory_space=pltpu.VMEM),
        scratch_shapes=[
            pltpu.SemaphoreType.DMA((3,)),
            pltpu.SemaphoreType.DMA((3,)),
            pltpu.SemaphoreType.REGULAR,
        ],
        compiler_params=pltpu.CompilerParams(collective_id=1),
    )(local_amax)
    return jnp.max(buf)


def _ag_body(q_r_ref, q_l_ref, qfull_ref, ag_send_r, ag_recv_r,
             ag_send_l, ag_recv_l, cp_sems, exit_sem):
    i = lax.axis_index("i")
    left = (i - 1) % N_DEV
    right = (i + 1) % N_DEV
    own_r = (i + 1) % N_DEV
    own_l = (i + 3) % N_DEV

    barrier = pltpu.get_barrier_semaphore()
    _neighbor_barrier(barrier, (left, right), 2)

    st_r = pltpu.make_async_copy(
        q_r_ref, qfull_ref.at[pl.ds(own_r * CM, CM), pl.ds(0, NL)],
        cp_sems.at[0])
    st_l = pltpu.make_async_copy(
        q_l_ref, qfull_ref.at[pl.ds(own_l * CM, CM), pl.ds(NL, NL)],
        cp_sems.at[1])
    st_r.start()
    st_l.start()

    for t in range(N_DEV - 1):
        g_r = (own_r - t) % N_DEV
        g_l = (own_l + t) % N_DEV
        src_r = (q_r_ref if t == 0
                 else qfull_ref.at[pl.ds(g_r * CM, CM), pl.ds(0, NL)])
        rdma_r = pltpu.make_async_remote_copy(
            src_ref=src_r,
            dst_ref=qfull_ref.at[pl.ds(g_r * CM, CM), pl.ds(0, NL)],
            send_sem=ag_send_r.at[t],
            recv_sem=ag_recv_r.at[t],
            device_id=(right,),
            device_id_type=pl.DeviceIdType.MESH,
        )
        src_l = (q_l_ref if t == 0
                 else qfull_ref.at[pl.ds(g_l * CM, CM), pl.ds(NL, NL)])
        rdma_l = pltpu.make_async_remote_copy(
            src_ref=src_l,
            dst_ref=qfull_ref.at[pl.ds(g_l * CM, CM), pl.ds(NL, NL)],
            send_sem=ag_send_l.at[t],
            recv_sem=ag_recv_l.at[t],
            device_id=(left,),
            device_id_type=pl.DeviceIdType.MESH,
        )
        rdma_r.start()
        rdma_l.start()
        rdma_r.wait()
        rdma_l.wait()

    st_r.wait()
    st_l.wait()

    _neighbor_barrier(exit_sem, (left, right), 2)


def _quant_allgather(q_r, q_l):
    return pl.pallas_call(
        _ag_body,
        out_shape=jax.ShapeDtypeStruct((M, N), q_r.dtype),
        in_specs=[
            pl.BlockSpec(memory_space=pltpu.HBM),
            pl.BlockSpec(memory_space=pltpu.HBM),
        ],
        out_specs=pl.BlockSpec(memory_space=pltpu.HBM),
        scratch_shapes=[
            pltpu.SemaphoreType.DMA((N_DEV - 1,)),
            pltpu.SemaphoreType.DMA((N_DEV - 1,)),
            pltpu.SemaphoreType.DMA((N_DEV - 1,)),
            pltpu.SemaphoreType.DMA((N_DEV - 1,)),
            pltpu.SemaphoreType.DMA((2,)),
            pltpu.SemaphoreType.REGULAR,
        ],
        compiler_params=pltpu.CompilerParams(collective_id=2),
    )(q_r, q_l)


def kernel(x, w_mat):
    own_r, own_l, local_amax = _reduce_scatter(
        x.astype(jnp.bfloat16), w_mat.astype(jnp.bfloat16))

    amax = _amax_allgather(local_amax)
    scale = amax / 448.0

    q_r = (own_r / scale).astype(jnp.float8_e4m3fn)
    q_l = (own_l / scale).astype(jnp.float8_e4m3fn)
    qfull = _quant_allgather(q_r, q_l)

    return (qfull.astype(jnp.float32) * scale).astype(jnp.bfloat16)


# device time: 718218 ns/iter; 2.3011x vs baseline; 1.1008x over previous
import jax
import jax.numpy as jnp
from jax import lax
from jax.experimental import pallas as pl
from jax.experimental.pallas import tpu as pltpu

N_DEV = 4
M = 4096
N = 8192
NL = N // 2
CM = M // N_DEV
TILE = 256
NT = CM // TILE


def _neighbor_barrier(sem, nbrs, count):
    for nbr in nbrs:
        pl.semaphore_signal(sem, inc=1, device_id=(nbr,),
                            device_id_type=pl.DeviceIdType.MESH)
    pl.semaphore_wait(sem, count)


KS = 1024


def _rs_body(x_ref, w_ref, pchunk_ref, own_r_ref, own_l_ref, amax_ref,
             recv_r_ref, recv_l_ref, recv0_r_ref, recv0_l_ref,
             w_vmem, x_tile, acc_r_ref, acc_l_ref, b_ref, b16_ref,
             rs_send_r, rs_recv_r, rs_send_l, rs_recv_l, st_sems, cp_sems,
             exit_sem):
    i = lax.axis_index("i")
    left = (i - 1) % N_DEV
    right = (i + 1) % N_DEV

    wcp = pltpu.make_async_copy(w_ref, w_vmem, cp_sems.at[3])
    wcp.start()

    barrier = pltpu.get_barrier_semaphore()
    _neighbor_barrier(barrier, (left, right), 2)

    wcp.wait()

    def _ts(t):
        return slice(t * TILE, (t + 1) * TILE)

    def _send_tile(acc_ref, recv_ref, send_sems, recv_sems, s, t, dev):
        r = pltpu.make_async_remote_copy(
            src_ref=acc_ref.at[_ts(t), :],
            dst_ref=recv_ref.at[s - 1, _ts(t), :],
            send_sem=send_sems.at[s * NT + t],
            recv_sem=recv_sems.at[s * NT + t],
            device_id=(dev,),
            device_id_type=pl.DeviceIdType.MESH,
        )
        r.start()
        return r

    def _send_tile0(col0, recv0_ref, send_sems, recv_sems, t, dev):
        r = pltpu.make_async_remote_copy(
            src_ref=pchunk_ref.at[_ts(t), pl.ds(col0, NL)],
            dst_ref=recv0_ref.at[_ts(t), :],
            send_sem=send_sems.at[t],
            recv_sem=recv_sems.at[t],
            device_id=(dev,),
            device_id_type=pl.DeviceIdType.MESH,
        )
        r.start()
        return r

    def _partial_tile(c, t, col0):
        xcp = pltpu.make_async_copy(
            x_ref.at[pl.ds(c * CM + t * TILE, TILE), :],
            x_tile, cp_sems.at[0])
        xcp.start()
        xcp.wait()
        return jnp.dot(x_tile[...], w_vmem[:, col0:col0 + NL],
                       preferred_element_type=jnp.float32)

    rd_r = [None] * NT
    rd_l = [None] * NT
    for t in range(NT):
        rd_r[t] = _send_tile0(0, recv0_r_ref, rs_send_r, rs_recv_r, t, right)
        rd_l[t] = _send_tile0(NL, recv0_l_ref, rs_send_l, rs_recv_l, t, left)

    amax = jnp.full((), 0.0, jnp.float32)
    sts = []

    for s in range(N_DEV - 1):
        c_r = (i - s - 1) % N_DEV
        c_l = (i + s + 1) % N_DEV
        final = s == N_DEV - 2
        for t in range(NT):
            rd_r[t].wait()
            if s == 0:
                cp_b = pltpu.make_async_copy(
                    recv0_r_ref.at[_ts(t), :], b16_ref, cp_sems.at[1])
            else:
                cp_b = pltpu.make_async_copy(
                    recv_r_ref.at[s - 1, _ts(t), :], b_ref, cp_sems.at[1])
            cp_b.start()
            v = _partial_tile(c_r, t, 0)
            cp_b.wait()
            v = v + (b16_ref[...].astype(jnp.float32) if s == 0
                     else b_ref[...])
            acc_r_ref[_ts(t), :] = v
            if final:
                amax = jnp.maximum(amax, jnp.max(jnp.abs(v)))
                st = pltpu.make_async_copy(
                    acc_r_ref.at[_ts(t), :], own_r_ref.at[_ts(t), :],
                    st_sems.at[t])
                st.start()
                sts.append(st)
            else:
                rd_r[t] = _send_tile(acc_r_ref, recv_r_ref, rs_send_r,
                                     rs_recv_r, s + 1, t, right)

            rd_l[t].wait()
            if s == 0:
                cp_b2 = pltpu.make_async_copy(
                    recv0_l_ref.at[_ts(t), :], b16_ref, cp_sems.at[2])
            else:
                cp_b2 = pltpu.make_async_copy(
                    recv_l_ref.at[s - 1, _ts(t), :], b_ref, cp_sems.at[2])
            cp_b2.start()
            v2 = _partial_tile(c_l, t, NL)
            cp_b2.wait()
            v2 = v2 + (b16_ref[...].astype(jnp.float32) if s == 0
                       else b_ref[...])
            acc_l_ref[_ts(t), :] = v2
            if final:
                amax = jnp.maximum(amax, jnp.max(jnp.abs(v2)))
                st = pltpu.make_async_copy(
                    acc_l_ref.at[_ts(t), :], own_l_ref.at[_ts(t), :],
                    st_sems.at[NT + t])
                st.start()
                sts.append(st)
            else:
                rd_l[t] = _send_tile(acc_l_ref, recv_l_ref, rs_send_l,
                                     rs_recv_l, s + 1, t, left)

    amax_ref[...] = jnp.full((8, 128), amax, jnp.float32)
    for st in sts:
        st.wait()

    _neighbor_barrier(exit_sem, (left, right), 2)


def _reduce_scatter(x_bf16, w_bf16, pchunk):
    own_r, own_l, amax, _, _, _, _ = pl.pallas_call(
        _rs_body,
        out_shape=(
            jax.ShapeDtypeStruct((CM, NL), jnp.float32),
            jax.ShapeDtypeStruct((CM, NL), jnp.float32),
            jax.ShapeDtypeStruct((8, 128), jnp.float32),
            jax.ShapeDtypeStruct((N_DEV - 2, CM, NL), jnp.float32),
            jax.ShapeDtypeStruct((N_DEV - 2, CM, NL), jnp.float32),
            jax.ShapeDtypeStruct((CM, NL), jnp.bfloat16),
            jax.ShapeDtypeStruct((CM, NL), jnp.bfloat16),
        ),
        in_specs=[
            pl.BlockSpec(memory_space=pltpu.HBM),
            pl.BlockSpec(memory_space=pltpu.HBM),
            pl.BlockSpec(memory_space=pltpu.HBM),
        ],
        out_specs=(
            pl.BlockSpec(memory_space=pltpu.HBM),
            pl.BlockSpec(memory_space=pltpu.HBM),
            pl.BlockSpec(memory_space=pltpu.VMEM),
            pl.BlockSpec(memory_space=pltpu.HBM),
            pl.BlockSpec(memory_space=pltpu.HBM),
            pl.BlockSpec(memory_space=pltpu.HBM),
            pl.BlockSpec(memory_space=pltpu.HBM),
        ),
        scratch_shapes=[
            pltpu.VMEM((KS, N), jnp.bfloat16),
            pltpu.VMEM((TILE, KS), jnp.bfloat16),
            pltpu.VMEM((CM, NL), jnp.float32),
            pltpu.VMEM((CM, NL), jnp.float32),
            pltpu.VMEM((TILE, NL), jnp.float32),
            pltpu.VMEM((TILE, NL), jnp.bfloat16),
            pltpu.SemaphoreType.DMA(((N_DEV - 1) * NT,)),
            pltpu.SemaphoreType.DMA(((N_DEV - 1) * NT,)),
            pltpu.SemaphoreType.DMA(((N_DEV - 1) * NT,)),
            pltpu.SemaphoreType.DMA(((N_DEV - 1) * NT,)),
            pltpu.SemaphoreType.DMA((2 * NT,)),
            pltpu.SemaphoreType.DMA((4,)),
            pltpu.SemaphoreType.REGULAR,
        ],
        compiler_params=pltpu.CompilerParams(
            collective_id=0, vmem_limit_bytes=63 * 1024 * 1024),
    )(x_bf16, w_bf16, pchunk)
    return own_r, own_l, amax


def _amax_body(amax_ref, buf_ref, ax_send, ax_recv, exit_sem):
    i = lax.axis_index("i")
    peers = [(i + d) % N_DEV for d in (1, 2, 3)]

    barrier = pltpu.get_barrier_semaphore()
    _neighbor_barrier(barrier, peers, 3)

    buf_ref[pl.ds(i * 8, 8), :] = amax_ref[...]

    rdmas = []
    for d in (1, 2, 3):
        rdma = pltpu.make_async_remote_copy(
            src_ref=buf_ref.at[pl.ds(i * 8, 8), :],
            dst_ref=buf_ref.at[pl.ds(i * 8, 8), :],
            send_sem=ax_send.at[d - 1],
            recv_sem=ax_recv.at[d - 1],
            device_id=(peers[d - 1],),
            device_id_type=pl.DeviceIdType.MESH,
        )
        rdma.start()
        rdmas.append(rdma)
    for rdma in rdmas:
        rdma.wait()

    _neighbor_barrier(exit_sem, peers, 3)


def _amax_allgather(local_amax):
    buf = pl.pallas_call(
        _amax_body,
        out_shape=jax.ShapeDtypeStruct((N_DEV * 8, 128), jnp.float32),
        in_specs=[pl.BlockSpec(memory_space=pltpu.VMEM)],
        out_specs=pl.BlockSpec(memory_space=pltpu.VMEM),
        scratch_shapes=[
            pltpu.SemaphoreType.DMA((3,)),
            pltpu.SemaphoreType.DMA((3,)),
            pltpu.SemaphoreType.REGULAR,
        ],
        compiler_params=pltpu.CompilerParams(collective_id=1),
    )(local_amax)
    return jnp.max(buf)


def _ag_body(q_r_ref, q_l_ref, qfull_ref, ag_send_r, ag_recv_r,
             ag_send_l, ag_recv_l, cp_sems, exit_sem):
    i = lax.axis_index("i")
    left = (i - 1) % N_DEV
    right = (i + 1) % N_DEV
    own_r = (i + 1) % N_DEV
    own_l = (i + 3) % N_DEV

    barrier = pltpu.get_barrier_semaphore()
    _neighbor_barrier(barrier, (left, right), 2)

    st_r = pltpu.make_async_copy(
        q_r_ref, qfull_ref.at[pl.ds(own_r * CM, CM), pl.ds(0, NL)],
        cp_sems.at[0])
    st_l = pltpu.make_async_copy(
        q_l_ref, qfull_ref.at[pl.ds(own_l * CM, CM), pl.ds(NL, NL)],
        cp_sems.at[1])
    st_r.start()
    st_l.start()

    for t in range(N_DEV - 1):
        g_r = (own_r - t) % N_DEV
        g_l = (own_l + t) % N_DEV
        src_r = (q_r_ref if t == 0
                 else qfull_ref.at[pl.ds(g_r * CM, CM), pl.ds(0, NL)])
        rdma_r = pltpu.make_async_remote_copy(
            src_ref=src_r,
            dst_ref=qfull_ref.at[pl.ds(g_r * CM, CM), pl.ds(0, NL)],
            send_sem=ag_send_r.at[t],
            recv_sem=ag_recv_r.at[t],
            device_id=(right,),
            device_id_type=pl.DeviceIdType.MESH,
        )
        src_l = (q_l_ref if t == 0
                 else qfull_ref.at[pl.ds(g_l * CM, CM), pl.ds(NL, NL)])
        rdma_l = pltpu.make_async_remote_copy(
            src_ref=src_l,
            dst_ref=qfull_ref.at[pl.ds(g_l * CM, CM), pl.ds(NL, NL)],
            send_sem=ag_send_l.at[t],
            recv_sem=ag_recv_l.at[t],
            device_id=(left,),
            device_id_type=pl.DeviceIdType.MESH,
        )
        rdma_r.start()
        rdma_l.start()
        rdma_r.wait()
        rdma_l.wait()

    st_r.wait()
    st_l.wait()

    _neighbor_barrier(exit_sem, (left, right), 2)


def _quant_allgather(q_r, q_l):
    return pl.pallas_call(
        _ag_body,
        out_shape=jax.ShapeDtypeStruct((M, N), q_r.dtype),
        in_specs=[
            pl.BlockSpec(memory_space=pltpu.HBM),
            pl.BlockSpec(memory_space=pltpu.HBM),
        ],
        out_specs=pl.BlockSpec(memory_space=pltpu.HBM),
        scratch_shapes=[
            pltpu.SemaphoreType.DMA((N_DEV - 1,)),
            pltpu.SemaphoreType.DMA((N_DEV - 1,)),
            pltpu.SemaphoreType.DMA((N_DEV - 1,)),
            pltpu.SemaphoreType.DMA((N_DEV - 1,)),
            pltpu.SemaphoreType.DMA((2,)),
            pltpu.SemaphoreType.REGULAR,
        ],
        compiler_params=pltpu.CompilerParams(collective_id=2),
    )(q_r, q_l)


def kernel(x, w_mat):
    xb = x.astype(jnp.bfloat16)
    wb = w_mat.astype(jnp.bfloat16)

    i = lax.axis_index("i")
    x_rows = lax.dynamic_slice(xb, (i * CM, 0), (CM, KS))
    pchunk = jnp.dot(
        x_rows, wb, preferred_element_type=jnp.float32
    ).astype(jnp.bfloat16)

    own_r, own_l, local_amax = _reduce_scatter(xb, wb, pchunk)

    amax = _amax_allgather(local_amax)
    scale = amax / 448.0

    q_r = (own_r / scale).astype(jnp.float8_e4m3fn)
    q_l = (own_l / scale).astype(jnp.float8_e4m3fn)
    qfull = _quant_allgather(q_r, q_l)

    return (qfull.astype(jnp.float32) * scale).astype(jnp.bfloat16)
